# baseline (speedup 1.0000x reference)
import functools

import jax
import jax.numpy as jnp
from jax import lax
from jax.experimental import pallas as pl
from jax.experimental.pallas import tpu as pltpu
from jax.experimental.pallas import tpu_sc as plsc

_BATCH = 16384
_DIM = 64
_NC, _NS = 2, 16
_NW = _NC * _NS
_PER_W = _BATCH // _NW
_CHUNK = 128
_NCHUNK = _PER_W // _CHUNK

_mesh = plsc.VectorSubcoreMesh(core_axis_name="c", subcore_axis_name="s")


@functools.partial(
    pl.kernel,
    mesh=_mesh,
    out_type=jax.ShapeDtypeStruct((_BATCH, 3, _DIM), jnp.float32),
    scratch_types=[
        pltpu.VMEM((_NCHUNK, _CHUNK), jnp.int32),
        pltpu.VMEM((_NCHUNK, _CHUNK), jnp.int32),
        pltpu.VMEM((_NCHUNK, _CHUNK), jnp.int32),
        pltpu.VMEM((_PER_W, _DIM), jnp.float32),
        pltpu.VMEM((_PER_W, _DIM), jnp.float32),
        pltpu.VMEM((_PER_W, _DIM), jnp.float32),
        pltpu.SemaphoreType.DMA,
    ],
    compiler_params=pltpu.CompilerParams(use_tc_tiling_on_sc=False),
)
def _gather_kernel(h_hbm, r_hbm, t_hbm, ent_hbm, rel_hbm, out_hbm,
                   hi, ri, ti, hv, rv, tv, sem):
    wid = lax.axis_index("s") * _NC + lax.axis_index("c")
    row0 = wid * _NCHUNK
    pltpu.sync_copy(h_hbm.at[pl.ds(row0, _NCHUNK)], hi)
    pltpu.sync_copy(r_hbm.at[pl.ds(row0, _NCHUNK)], ri)
    pltpu.sync_copy(t_hbm.at[pl.ds(row0, _NCHUNK)], ti)
    cps = []
    for j in range(_NCHUNK):
        dst = pl.ds(j * _CHUNK, _CHUNK)
        cps.append(pltpu.async_copy(ent_hbm.at[hi.at[j]], hv.at[dst], sem))
        cps.append(pltpu.async_copy(rel_hbm.at[ri.at[j]], rv.at[dst], sem))
        cps.append(pltpu.async_copy(ent_hbm.at[ti.at[j]], tv.at[dst], sem))
    for cp in cps:
        cp.wait()
    base = pl.ds(wid * _PER_W, _PER_W)
    pltpu.sync_copy(hv, out_hbm.at[base, 0])
    pltpu.sync_copy(rv, out_hbm.at[base, 1])
    pltpu.sync_copy(tv, out_hbm.at[base, 2])


def kernel(inputs, entity_table, relation_table):
    idx = inputs.astype(jnp.int32)
    heads = idx[:, 0].reshape(-1, _CHUNK)
    rels = idx[:, 1].reshape(-1, _CHUNK)
    tails = idx[:, 2].reshape(-1, _CHUNK)
    return _gather_kernel(heads, rels, tails, entity_table, relation_table)
